# baseline (speedup 1.0000x reference)
import functools

import jax
import jax.numpy as jnp
from jax import lax
from jax.experimental import pallas as pl
from jax.experimental.pallas import tpu as pltpu
from jax.experimental.pallas import tpu_sc as plsc

VOCAB = 1000000
DIM = 64
SEQ_LEN = 200
BATCH = 4096

NUM_CORES = 2
NUM_SUBCORES = 16
NW = NUM_CORES * NUM_SUBCORES
HALF = 245 * 2048
BC = 256
CHUNKS_PER_S = BATCH // BC
UNITS = SEQ_LEN * CHUNKS_PER_S
UNITS_PER_W = UNITS // NW
PAIRS = UNITS_PER_W // 2
LANES = 16
GROUPS = BC // LANES


def _pos_encoding():
    even_i = jnp.arange(0, DIM, 2).astype(jnp.float32)
    denominator = jnp.power(10000.0, even_i / DIM)
    position = jnp.arange(SEQ_LEN).reshape(SEQ_LEN, 1).astype(jnp.float32)
    even_pe = jnp.sin(position / denominator)
    odd_pe = jnp.cos(position / denominator)
    return jnp.stack([even_pe, odd_pe], axis=2).reshape(SEQ_LEN, DIM)


def _prep_body(lo_ref, hi_ref, out_ref):
    eye = (lax.broadcasted_iota(jnp.int32, (DIM, DIM), 0) ==
           lax.broadcasted_iota(jnp.int32, (DIM, DIM), 1)).astype(jnp.float32)
    dn = (((0,), (0,)), ((), ()))
    lo = lax.dot_general(lo_ref[...], eye, dn,
                         preferred_element_type=jnp.float32)
    hi = lax.dot_general(hi_ref[...], eye, dn,
                         preferred_element_type=jnp.float32)
    out_ref[...] = jnp.concatenate([lo, hi], axis=1)


_prep_call = pl.pallas_call(
    _prep_body,
    grid=(245,),
    in_specs=[
        pl.BlockSpec((64, 2048), lambda i: (0, i)),
        pl.BlockSpec((64, 2048), lambda i: (0, jnp.minimum(i + 245, 488))),
    ],
    out_specs=pl.BlockSpec((2048, 2 * DIM), lambda i: (i, 0)),
    out_shape=jax.ShapeDtypeStruct((HALF, 2 * DIM), jnp.float32),
)


def _body(gidx_hbm, tab_hbm, pe_hbm, out_hbm,
          idx0, idx1, g0, g1, o0, o1, pe_v,
          gsem0, gsem1, wsem0, wsem1):
    wid = lax.axis_index("s") * NUM_CORES + lax.axis_index("c")
    pltpu.sync_copy(pe_hbm, pe_v)
    u0 = wid * UNITS_PER_W

    def unit_su(u):
        s = lax.shift_right_logical(u, 4)
        b0 = lax.mul(lax.bitwise_and(u, CHUNKS_PER_S - 1), BC)
        return s, b0

    def stage(u, idx_v):
        s, b0 = unit_su(u)
        for j in range(BC // 128):
            pltpu.sync_copy(gidx_hbm.at[s, pl.ds(b0 + j * 128, 128)],
                            idx_v.at[j])

    def fire_gathers(idx_v, g_v, gsem):
        for j in range(BC // 128):
            pltpu.make_async_copy(
                tab_hbm.at[idx_v.at[j]],
                g_v.at[pl.ds(j * 128, 128)],
                gsem,
            ).start()

    def wait_gathers(idx_v, g_v, gsem):
        for j in range(BC // 128):
            pltpu.make_async_copy(
                tab_hbm.at[idx_v.at[j]],
                g_v.at[pl.ds(j * 128, 128)],
                gsem,
            ).wait()

    def scatter_rows(u, g_v, o_v):
        s, _ = unit_su(u)
        pe = [pe_v[s, pl.ds(dd * LANES, LANES)] for dd in range(DIM // LANES)]
        dvs = [lax.iota(jnp.int32, LANES) + dd * LANES
               for dd in range(DIM // LANES)]

        def g_body(g, carry):
            for j in range(LANES):
                r = g * LANES + j
                bv = jnp.broadcast_to(r, (LANES,))
                for dd in range(DIM // LANES):
                    val = g_v[r, pl.ds(dd * LANES, LANES)] + pe[dd]
                    plsc.store_scatter(o_v, [dvs[dd], bv], val)
            return carry

        lax.fori_loop(0, GROUPS, g_body, 0)

    def fire_wb(u, o_v, wsem):
        s, b0 = unit_su(u)
        pltpu.make_async_copy(
            o_v, out_hbm.at[pl.ds(s * DIM, DIM), pl.ds(b0, BC)], wsem
        ).start()

    def drain_wb(o_v, wsem):
        pltpu.make_async_copy(
            o_v, out_hbm.at[pl.ds(0, DIM), pl.ds(0, BC)], wsem
        ).wait()

    stage(u0, idx0)
    fire_gathers(idx0, g0, gsem0)

    def pair_body(k, carry):
        u = u0 + 2 * k
        stage(u + 1, idx1)
        fire_gathers(idx1, g1, gsem1)
        wait_gathers(idx0, g0, gsem0)

        @pl.when(k >= 1)
        def _():
            drain_wb(o0, wsem0)
        scatter_rows(u, g0, o0)
        fire_wb(u, o0, wsem0)

        @pl.when(k < PAIRS - 1)
        def _():
            stage(u + 2, idx0)
            fire_gathers(idx0, g0, gsem0)
        wait_gathers(idx1, g1, gsem1)

        @pl.when(k >= 1)
        def _():
            drain_wb(o1, wsem1)
        scatter_rows(u + 1, g1, o1)
        fire_wb(u + 1, o1, wsem1)
        return carry

    lax.fori_loop(0, PAIRS, pair_body, 0)
    drain_wb(o0, wsem0)
    drain_wb(o1, wsem1)


_emb_call = functools.partial(
    pl.kernel,
    mesh=plsc.VectorSubcoreMesh(core_axis_name="c", subcore_axis_name="s"),
    out_type=jax.ShapeDtypeStruct((SEQ_LEN * DIM, BATCH), jnp.float32),
    scratch_types=[
        pltpu.VMEM((BC // 128, 128), jnp.int32),
        pltpu.VMEM((BC // 128, 128), jnp.int32),
        pltpu.VMEM((BC, DIM), jnp.float32),
        pltpu.VMEM((BC, DIM), jnp.float32),
        pltpu.VMEM((DIM, BC), jnp.float32),
        pltpu.VMEM((DIM, BC), jnp.float32),
        pltpu.VMEM((SEQ_LEN, DIM), jnp.float32),
        pltpu.SemaphoreType.DMA,
        pltpu.SemaphoreType.DMA,
        pltpu.SemaphoreType.DMA,
        pltpu.SemaphoreType.DMA,
    ],
    compiler_params=pltpu.CompilerParams(
        use_tc_tiling_on_sc=False, needs_layout_passes=False),
)(_body)


@jax.jit
def kernel(inputs, word_embedding):
    idx_t = inputs.astype(jnp.int32).T
    hi = idx_t >= HALF
    gidx = jnp.where(hi, 2 * (idx_t - HALF) + 1, 2 * idx_t)
    wbt = word_embedding.T
    packed = _prep_call(wbt, wbt)
    tab = packed.reshape(2 * HALF, DIM)
    pe = _pos_encoding()
    out_sdb = _emb_call(gidx, tab, pe)
    return jnp.transpose(out_sdb.reshape(SEQ_LEN, DIM, BATCH), (2, 0, 1))


# speedup vs baseline: 1.7891x; 1.7891x over previous
import functools

import jax
import jax.numpy as jnp
from jax import lax
from jax.experimental import pallas as pl
from jax.experimental.pallas import tpu as pltpu
from jax.experimental.pallas import tpu_sc as plsc

VOCAB = 1000000
DIM = 64
SEQ_LEN = 200
BATCH = 4096

NUM_CORES = 2
NUM_SUBCORES = 16
NW = NUM_CORES * NUM_SUBCORES
HALF = 245 * 2048
BC = 256
CHUNKS_PER_S = BATCH // BC
UNITS = SEQ_LEN * CHUNKS_PER_S
UNITS_PER_W = UNITS // NW
PAIRS = UNITS_PER_W // 2
LANES = 16
GROUPS = BC // LANES


def _pos_encoding():
    even_i = jnp.arange(0, DIM, 2).astype(jnp.float32)
    denominator = jnp.power(10000.0, even_i / DIM)
    position = jnp.arange(SEQ_LEN).reshape(SEQ_LEN, 1).astype(jnp.float32)
    even_pe = jnp.sin(position / denominator)
    odd_pe = jnp.cos(position / denominator)
    return jnp.stack([even_pe, odd_pe], axis=2).reshape(SEQ_LEN, DIM)


def _prep_body(lo_ref, hi_ref, out_ref):
    eye = (lax.broadcasted_iota(jnp.int32, (DIM, DIM), 0) ==
           lax.broadcasted_iota(jnp.int32, (DIM, DIM), 1)).astype(jnp.float32)
    dn = (((0,), (0,)), ((), ()))
    lo = lax.dot_general(lo_ref[...], eye, dn,
                         preferred_element_type=jnp.float32)
    hi = lax.dot_general(hi_ref[...], eye, dn,
                         preferred_element_type=jnp.float32)
    out_ref[...] = jnp.concatenate([lo, hi], axis=1)


_prep_call = pl.pallas_call(
    _prep_body,
    grid=(245,),
    in_specs=[
        pl.BlockSpec((64, 2048), lambda i: (0, i)),
        pl.BlockSpec((64, 2048), lambda i: (0, jnp.minimum(i + 245, 488))),
    ],
    out_specs=pl.BlockSpec((2048, 2 * DIM), lambda i: (i, 0)),
    out_shape=jax.ShapeDtypeStruct((HALF, 2 * DIM), jnp.float32),
)


def _body(gidx_hbm, tab_hbm, pe_hbm, out_hbm,
          idx0, idx1, g0, g1, pe_v,
          gsem0, gsem1, wsem0, wsem1):
    wid = lax.axis_index("s") * NUM_CORES + lax.axis_index("c")
    pltpu.sync_copy(pe_hbm, pe_v)
    u0 = wid * UNITS_PER_W

    def unit_su(u):
        s = lax.shift_right_logical(u, 4)
        b0 = lax.mul(lax.bitwise_and(u, CHUNKS_PER_S - 1), BC)
        return s, b0

    def stage(u, idx_v):
        s, b0 = unit_su(u)
        for j in range(BC // 128):
            pltpu.sync_copy(gidx_hbm.at[s, pl.ds(b0 + j * 128, 128)],
                            idx_v.at[j])

    def fire_gathers(idx_v, g_v, gsem):
        for j in range(BC // 128):
            pltpu.make_async_copy(
                tab_hbm.at[idx_v.at[j]],
                g_v.at[pl.ds(j * 128, 128)],
                gsem,
            ).start()

    def wait_gathers(idx_v, g_v, gsem):
        for j in range(BC // 128):
            pltpu.make_async_copy(
                tab_hbm.at[idx_v.at[j]],
                g_v.at[pl.ds(j * 128, 128)],
                gsem,
            ).wait()

    def add_pe(u, g_v):
        s, _ = unit_su(u)
        pe = [pe_v[s, pl.ds(dd * LANES, LANES)] for dd in range(DIM // LANES)]

        def g_body(g, carry):
            base = g * LANES
            for j in range(LANES):
                r = base + j
                for dd in range(DIM // LANES):
                    sl = pl.ds(dd * LANES, LANES)
                    g_v[r, sl] = g_v[r, sl] + pe[dd]
            return carry

        lax.fori_loop(0, GROUPS, g_body, 0)

    def fire_wb(u, g_v, wsem):
        s, b0 = unit_su(u)
        pltpu.make_async_copy(
            g_v, out_hbm.at[s, pl.ds(b0, BC)], wsem
        ).start()

    def drain_wb(g_v, wsem):
        pltpu.make_async_copy(
            g_v, out_hbm.at[0, pl.ds(0, BC)], wsem
        ).wait()

    stage(u0, idx0)
    fire_gathers(idx0, g0, gsem0)

    def pair_body(k, carry):
        u = u0 + 2 * k
        stage(u + 1, idx1)

        @pl.when(k >= 1)
        def _():
            drain_wb(g1, wsem1)
        fire_gathers(idx1, g1, gsem1)
        wait_gathers(idx0, g0, gsem0)
        add_pe(u, g0)
        fire_wb(u, g0, wsem0)

        @pl.when(k < PAIRS - 1)
        def _():
            stage(u + 2, idx0)
            drain_wb(g0, wsem0)
            fire_gathers(idx0, g0, gsem0)
        wait_gathers(idx1, g1, gsem1)
        add_pe(u + 1, g1)
        fire_wb(u + 1, g1, wsem1)
        return carry

    lax.fori_loop(0, PAIRS, pair_body, 0)
    drain_wb(g0, wsem0)
    drain_wb(g1, wsem1)


_emb_call = functools.partial(
    pl.kernel,
    mesh=plsc.VectorSubcoreMesh(core_axis_name="c", subcore_axis_name="s"),
    out_type=jax.ShapeDtypeStruct((SEQ_LEN, BATCH, DIM), jnp.float32),
    scratch_types=[
        pltpu.VMEM((BC // 128, 128), jnp.int32),
        pltpu.VMEM((BC // 128, 128), jnp.int32),
        pltpu.VMEM((BC, DIM), jnp.float32),
        pltpu.VMEM((BC, DIM), jnp.float32),
        pltpu.VMEM((SEQ_LEN, DIM), jnp.float32),
        pltpu.SemaphoreType.DMA,
        pltpu.SemaphoreType.DMA,
        pltpu.SemaphoreType.DMA,
        pltpu.SemaphoreType.DMA,
    ],
    compiler_params=pltpu.CompilerParams(
        use_tc_tiling_on_sc=False, needs_layout_passes=False),
)(_body)


@jax.jit
def kernel(inputs, word_embedding):
    idx_t = inputs.astype(jnp.int32).T
    hi = idx_t >= HALF
    gidx = jnp.where(hi, 2 * (idx_t - HALF) + 1, 2 * idx_t)
    wbt = word_embedding.T
    packed = _prep_call(wbt, wbt)
    tab = packed.reshape(2 * HALF, DIM)
    pe = _pos_encoding()
    out_sbd = _emb_call(gidx, tab, pe)
    return jnp.transpose(out_sbd, (1, 0, 2))
